# Initial kernel scaffold; baseline (speedup 1.0000x reference)
import jax
import jax.numpy as jnp
from jax.experimental import pallas as pl


def kernel(x, table, W1, b1, W2, b2):
    raise NotImplementedError("write your pallas kernel here")


# speedup vs baseline: 17.7753x; 17.7753x over previous
import functools

import jax
import jax.numpy as jnp
from jax import lax
from jax.experimental import pallas as pl
from jax.experimental.pallas import tpu as pltpu
from jax.experimental.pallas import tpu_sc as plsc

VOCAB = 100000
DIM = 128
B = 16384
L = 200
HID = 50
OUT = 4

NC = 2
NS = 16
NW = NC * NS
SEQ_PER_W = B // NW
GROUP = 64
NGROUP = SEQ_PER_W // GROUP
VPR = DIM // 16


def _sc_body(x_hbm, table_hbm, sums_hbm, idx_v, rows0, rows1, out_v, sem0, sem1):
    wid = lax.axis_index("s") * NC + lax.axis_index("c")
    seq0 = wid * SEQ_PER_W

    def issue(s_local, rows_ref, sem):
        base = s_local * L
        pltpu.async_copy(table_hbm.at[idx_v.at[pl.ds(base, 128)]],
                         rows_ref.at[pl.ds(0, 128)], sem)
        pltpu.async_copy(table_hbm.at[idx_v.at[pl.ds(base + 128, L - 128)]],
                         rows_ref.at[pl.ds(128, L - 128)], sem)

    def drain(rows_ref, sem):
        pltpu.make_async_copy(table_hbm.at[pl.ds(0, L)], rows_ref, sem).wait()

    def reduce(rows_ref, s_local):
        def body(r, acc):
            return tuple(acc[c] + rows_ref[r, pl.ds(16 * c, 16)]
                         for c in range(VPR))
        acc = lax.fori_loop(
            0, L, body,
            tuple(jnp.zeros((16,), jnp.float32) for _ in range(VPR)))
        for c in range(VPR):
            out_v[s_local, pl.ds(16 * c, 16)] = acc[c]

    @pl.loop(0, NGROUP)
    def _group(g):
        gseq = seq0 + g * GROUP
        pltpu.sync_copy(x_hbm.at[pl.ds(gseq * L, GROUP * L)], idx_v)
        issue(0, rows0, sem0)

        @pl.loop(0, GROUP, step=2)
        def _seq(s):
            issue(s + 1, rows1, sem1)
            drain(rows0, sem0)
            reduce(rows0, s)

            @pl.when(s + 2 < GROUP)
            def _():
                issue(s + 2, rows0, sem0)
            drain(rows1, sem1)
            reduce(rows1, s + 1)

        pltpu.sync_copy(out_v, sums_hbm.at[pl.ds(gseq, GROUP)])


_sc_sum = functools.partial(
    pl.kernel,
    out_type=jax.ShapeDtypeStruct((B, DIM), jnp.float32),
    mesh=plsc.VectorSubcoreMesh(core_axis_name="c", subcore_axis_name="s"),
    scratch_types=[
        pltpu.VMEM((GROUP * L,), jnp.int32),
        pltpu.VMEM((L, DIM), jnp.float32),
        pltpu.VMEM((L, DIM), jnp.float32),
        pltpu.VMEM((GROUP, DIM), jnp.float32),
        pltpu.SemaphoreType.DMA,
        pltpu.SemaphoreType.DMA,
    ],
)(_sc_body)


BLK = 2048


def _mlp_body(x_ref, sums_ref, w1_ref, b1_ref, w2_ref, b2_ref, out_ref):
    xb = x_ref[...]
    lengths = jnp.sum((xb != 0).astype(jnp.float32), axis=1, keepdims=True)
    pooled = sums_ref[...] / lengths
    h = jnp.dot(pooled, w1_ref[...], preferred_element_type=jnp.float32)
    h = jnp.maximum(h + b1_ref[...], 0.0)
    out_ref[...] = (jnp.dot(h, w2_ref[...], preferred_element_type=jnp.float32)
                    + b2_ref[...])


def _mlp(x2d, sums, w1, b1, w2, b2):
    return pl.pallas_call(
        _mlp_body,
        grid=(B // BLK,),
        in_specs=[
            pl.BlockSpec((BLK, L), lambda i: (i, 0)),
            pl.BlockSpec((BLK, DIM), lambda i: (i, 0)),
            pl.BlockSpec((DIM, HID), lambda i: (0, 0)),
            pl.BlockSpec((1, HID), lambda i: (0, 0)),
            pl.BlockSpec((HID, OUT), lambda i: (0, 0)),
            pl.BlockSpec((1, OUT), lambda i: (0, 0)),
        ],
        out_specs=pl.BlockSpec((BLK, OUT), lambda i: (i, 0)),
        out_shape=jax.ShapeDtypeStruct((B, OUT), jnp.float32),
    )(x2d, sums, w1, b1.reshape(1, HID), w2, b2.reshape(1, OUT))


def kernel(x, table, W1, b1, W2, b2):
    x32 = x.astype(jnp.int32)
    sums = _sc_sum(x32.reshape(B * L), table)
    return _mlp(x32, sums, W1, b1, W2, b2)
